# Initial kernel scaffold; baseline (speedup 1.0000x reference)
import jax
import jax.numpy as jnp
from jax.experimental import pallas as pl


def kernel(input_ids, table, W, b):
    raise NotImplementedError("write your pallas kernel here")


# speedup vs baseline: 2.9952x; 2.9952x over previous
import functools

import jax
import jax.numpy as jnp
from jax import lax
from jax.experimental import pallas as pl
from jax.experimental.pallas import tpu as pltpu
from jax.experimental.pallas import tpu_sc as plsc

LANES = 16


def _proj_body(t_ref, w_ref, o_ref):
    o_ref[...] = jnp.sum(t_ref[...] * w_ref[...], axis=1, keepdims=True)


def _project(table, Wt):
    V, E = table.shape
    BLK = 8192
    return pl.pallas_call(
        _proj_body,
        grid=(V // BLK,),
        in_specs=[
            pl.BlockSpec((BLK, E), lambda i: (i, 0)),
            pl.BlockSpec((1, E), lambda i: (0, 0)),
        ],
        out_specs=pl.BlockSpec((BLK, 1), lambda i: (i, 0)),
        out_shape=jax.ShapeDtypeStruct((V, 1), jnp.float32),
    )(table, Wt)


@functools.lru_cache(maxsize=None)
def _make_sc_pool(B, S):
    info = plsc.get_sparse_core_info()
    NC, NS = info.num_cores, info.num_subcores
    NW = NC * NS
    ROWS_W = B // NW
    GROUPS_W = ROWS_W // LANES
    CHUNK_G = 8
    CHUNK_IDX = CHUNK_G * LANES * S
    NCHUNK = GROUPS_W // CHUNK_G

    mesh = plsc.VectorSubcoreMesh(core_axis_name="c", subcore_axis_name="s")

    @functools.partial(
        pl.kernel,
        mesh=mesh,
        out_type=jax.ShapeDtypeStruct((B,), jnp.float32),
        scratch_types=[
            pltpu.VMEM((CHUNK_IDX,), jnp.int32),
            pltpu.VMEM((CHUNK_IDX,), jnp.float32),
            pltpu.VMEM((ROWS_W,), jnp.float32),
            pltpu.VMEM((LANES,), jnp.float32),
            pltpu.SemaphoreType.DMA,
        ],
    )
    def sc_pool(tw_hbm, ids_hbm, b8_hbm, out_hbm, idx_v, vals_v, out_v, b_v, sem):
        cid = lax.axis_index("c")
        sid = lax.axis_index("s")
        wid = sid * NC + cid
        pltpu.sync_copy(b8_hbm, b_v)
        bval = b_v[...]
        row0 = wid * ROWS_W
        inv_s = jnp.float32(1.0 / S)
        for c in range(NCHUNK):
            off = (row0 + c * CHUNK_G * LANES) * S
            pltpu.sync_copy(ids_hbm.at[pl.ds(off, CHUNK_IDX)], idx_v)
            pltpu.async_copy(tw_hbm.at[idx_v], vals_v, sem).wait()
            for g in range(CHUNK_G):
                base = g * LANES * S

                def t_body(t, acc, base=base):
                    return acc + vals_v[pl.ds(base + t * LANES, LANES)]

                acc = lax.fori_loop(0, S, t_body, jnp.zeros((LANES,), jnp.float32))
                r = acc * inv_s + bval
                out_v[pl.ds((c * CHUNK_G + g) * LANES, LANES)] = (
                    1.0 / (1.0 + jnp.exp(-r)))
        pltpu.sync_copy(out_v, out_hbm.at[pl.ds(row0, ROWS_W)])

    return sc_pool


def kernel(input_ids, table, W, b):
    B, S = input_ids.shape
    V, E = table.shape
    tw = _project(table, W.reshape(1, E)).reshape(V)
    ids_flat = (input_ids.astype(jnp.int32)
                .reshape(B // LANES, LANES, S)
                .transpose(0, 2, 1)
                .reshape(B * S))
    b16 = jnp.tile(b.astype(jnp.float32).reshape(1), LANES)
    out = _make_sc_pool(B, S)(tw, ids_flat, b16)
    return out.reshape(B, 1)
